# baseline (speedup 1.0000x reference)
import jax
import jax.numpy as jnp
from jax.experimental import pallas as pl
from jax.experimental.pallas import tpu as pltpu

_T, _D, _E, _F, _K = 2048, 1024, 8, 1024, 4
_BT = 2048


def _moe_body(x_ref, wg_ref, w1_ref, b1_ref, w2_ref, b2_ref, out_ref, g_ref):
    e = pl.program_id(1)
    xb = x_ref[...]

    @pl.when(e == 0)
    def _():
        logits = jax.lax.dot_general(
            xb, wg_ref[...], (((1,), (0,)), ((), ())),
            preferred_element_type=jnp.float32)
        col = jax.lax.broadcasted_iota(jnp.int32, (_BT, _E), 1)
        cnt = jnp.zeros((_BT, _E), jnp.float32)
        for e2 in range(_E):
            l2 = logits[:, e2:e2 + 1]
            beats = (l2 > logits) | ((l2 == logits) & (e2 < col))
            cnt += beats.astype(jnp.float32)
        sel = cnt < float(_K)
        m = jnp.max(logits, axis=1, keepdims=True)
        z = jnp.where(sel, jnp.exp(logits - m), 0.0)
        g_ref[...] = z / jnp.sum(z, axis=1, keepdims=True)

    h = jnp.dot(xb, w1_ref[0], preferred_element_type=jnp.float32)
    h = jnp.maximum(h + b1_ref[0], 0.0)
    o = jnp.dot(h, w2_ref[0], preferred_element_type=jnp.float32)
    o = o + b2_ref[0]
    col = jax.lax.broadcasted_iota(jnp.int32, (_BT, _E), 1)
    ge = jnp.sum(jnp.where(col == e, g_ref[...], 0.0), axis=1, keepdims=True)
    contrib = ge * o

    @pl.when(e == 0)
    def _():
        out_ref[...] = contrib

    @pl.when(e != 0)
    def _():
        out_ref[...] += contrib


def _moe_block(x, wg, w1, b1, w2, b2):
    return pl.pallas_call(
        _moe_body,
        grid=(_T // _BT, _E),
        in_specs=[
            pl.BlockSpec((_BT, _D), lambda i, e: (i, 0)),
            pl.BlockSpec((_D, _E), lambda i, e: (0, 0)),
            pl.BlockSpec((1, _D, _F), lambda i, e: (e, 0, 0)),
            pl.BlockSpec((1, 1, _F), lambda i, e: (e, 0, 0)),
            pl.BlockSpec((1, _F, _D), lambda i, e: (e, 0, 0)),
            pl.BlockSpec((1, 1, _D), lambda i, e: (e, 0, 0)),
        ],
        out_specs=pl.BlockSpec((_BT, _D), lambda i, e: (i, 0)),
        out_shape=jax.ShapeDtypeStruct((_T, _D), jnp.float32),
        scratch_shapes=[pltpu.VMEM((_BT, _E), jnp.float32)],
        compiler_params=pltpu.CompilerParams(
            dimension_semantics=("arbitrary", "arbitrary")),
    )(x, wg, w1, b1.reshape(_E, 1, _F), w2,
      b2.reshape(_E, 1, _D))


@jax.jit
def kernel(x, Wg0, W1_0, b1_0, W2_0, b2_0, Wg1, W1_1, b1_1, W2_1, b2_1):
    h = _moe_block(x, Wg0, W1_0, b1_0, W2_0, b2_0)
    return _moe_block(h, Wg1, W1_1, b1_1, W2_1, b2_1)


# speedup vs baseline: 1.0012x; 1.0012x over previous
import jax
import jax.numpy as jnp
from jax.experimental import pallas as pl
from jax.experimental.pallas import tpu as pltpu

_T, _D, _E, _F, _K = 2048, 1024, 8, 1024, 4
_BT = 1024


def _moe_body(x_ref, wg_ref, w1_ref, b1_ref, w2_ref, b2_ref, out_ref,
              g_ref, xb_ref, h_ref):
    ep = pl.program_id(1)

    @pl.when(ep == 0)
    def _():
        xb = x_ref[...].astype(jnp.bfloat16)
        xb_ref[...] = xb
        logits = jax.lax.dot_general(
            xb, wg_ref[...], (((1,), (0,)), ((), ())),
            preferred_element_type=jnp.float32)
        col = jax.lax.broadcasted_iota(jnp.int32, (_BT, _E), 1)
        cnt = jnp.zeros((_BT, _E), jnp.float32)
        for e2 in range(_E):
            l2 = logits[:, e2:e2 + 1]
            beats = (l2 > logits) | ((l2 == logits) & (e2 < col))
            cnt += beats.astype(jnp.float32)
        sel = cnt < float(_K)
        m = jnp.max(logits, axis=1, keepdims=True)
        z = jnp.where(sel, jnp.exp(logits - m), 0.0)
        g_ref[...] = z / jnp.sum(z, axis=1, keepdims=True)

    @pl.when(ep >= 1)
    def _():
        e = ep - 1
        h = jnp.maximum(h_ref[...] + b1_ref[0], 0.0)
        o = jnp.dot(h.astype(jnp.bfloat16),
                    w2_ref[0].astype(jnp.bfloat16),
                    preferred_element_type=jnp.float32)
        o = o + b2_ref[0]
        col = jax.lax.broadcasted_iota(jnp.int32, (_BT, _E), 1)
        ge = jnp.sum(jnp.where(col == e, g_ref[...], 0.0),
                     axis=1, keepdims=True)
        contrib = ge * o

        @pl.when(ep == 1)
        def _():
            out_ref[...] = contrib

        @pl.when(ep > 1)
        def _():
            out_ref[...] += contrib

    @pl.when(ep < _E)
    def _():
        h_ref[...] = jnp.dot(xb_ref[...],
                             w1_ref[0].astype(jnp.bfloat16),
                             preferred_element_type=jnp.float32)


def _moe_block(x, wg, w1, b1, w2, b2):
    def _w1_idx(i, ep):
        return (jnp.minimum(ep, _E - 1), 0, 0)

    def _prev_idx(i, ep):
        return (jnp.maximum(ep - 1, 0), 0, 0)

    return pl.pallas_call(
        _moe_body,
        grid=(_T // _BT, _E + 1),
        in_specs=[
            pl.BlockSpec((_BT, _D), lambda i, ep: (i, 0)),
            pl.BlockSpec((_D, _E), lambda i, ep: (0, 0)),
            pl.BlockSpec((1, _D, _F), _w1_idx),
            pl.BlockSpec((1, 1, _F), _prev_idx),
            pl.BlockSpec((1, _F, _D), _prev_idx),
            pl.BlockSpec((1, 1, _D), _prev_idx),
        ],
        out_specs=pl.BlockSpec((_BT, _D), lambda i, ep: (i, 0)),
        out_shape=jax.ShapeDtypeStruct((_T, _D), jnp.float32),
        scratch_shapes=[
            pltpu.VMEM((_BT, _E), jnp.float32),
            pltpu.VMEM((_BT, _D), jnp.bfloat16),
            pltpu.VMEM((_BT, _F), jnp.float32),
        ],
        compiler_params=pltpu.CompilerParams(
            dimension_semantics=("arbitrary", "arbitrary")),
    )(x, wg.astype(jnp.bfloat16), w1, b1.reshape(_E, 1, _F), w2,
      b2.reshape(_E, 1, _D))


@jax.jit
def kernel(x, Wg0, W1_0, b1_0, W2_0, b2_0, Wg1, W1_1, b1_1, W2_1, b2_1):
    h = _moe_block(x, Wg0, W1_0, b1_0, W2_0, b2_0)
    return _moe_block(h, Wg1, W1_1, b1_1, W2_1, b2_1)


# speedup vs baseline: 1.0655x; 1.0641x over previous
import jax
import jax.numpy as jnp
from jax.experimental import pallas as pl
from jax.experimental.pallas import tpu as pltpu

_T, _D, _E, _F, _K = 2048, 1024, 8, 1024, 4
_BT = 2048


def _moe_body(x_ref, wg_ref, w1_ref, b1_ref, w2_ref, b2_ref, out_ref, g_ref):
    e = pl.program_id(1)
    xb = x_ref[...].astype(jnp.bfloat16)

    @pl.when(e == 0)
    def _():
        logits = jax.lax.dot_general(
            xb, wg_ref[...], (((1,), (0,)), ((), ())),
            preferred_element_type=jnp.float32)
        lt = jnp.transpose(logits)
        row = jax.lax.broadcasted_iota(jnp.int32, (_E, _BT), 0)
        cnt = jnp.zeros((_E, _BT), jnp.float32)
        for e2 in range(_E):
            l2 = lt[e2:e2 + 1, :]
            beats = (l2 > lt) | ((l2 == lt) & (e2 < row))
            cnt += beats.astype(jnp.float32)
        sel = cnt < float(_K)
        m = jnp.max(lt, axis=0, keepdims=True)
        z = jnp.where(sel, jnp.exp(lt - m), 0.0)
        gt = z / jnp.sum(z, axis=0, keepdims=True)
        g_ref[...] = jnp.transpose(gt)

    h = jnp.dot(xb, w1_ref[0].astype(jnp.bfloat16),
                preferred_element_type=jnp.float32)
    h = jnp.maximum(h + b1_ref[0], 0.0)
    o = jnp.dot(h.astype(jnp.bfloat16), w2_ref[0].astype(jnp.bfloat16),
                preferred_element_type=jnp.float32)
    o = o + b2_ref[0]
    col = jax.lax.broadcasted_iota(jnp.int32, (_BT, _E), 1)
    ge = jnp.sum(jnp.where(col == e, g_ref[...], 0.0), axis=1, keepdims=True)
    contrib = ge * o

    @pl.when(e == 0)
    def _():
        out_ref[...] = contrib

    @pl.when(e != 0)
    def _():
        out_ref[...] += contrib


def _moe_block(x, wg, w1, b1, w2, b2):
    return pl.pallas_call(
        _moe_body,
        grid=(_T // _BT, _E),
        in_specs=[
            pl.BlockSpec((_BT, _D), lambda i, e: (i, 0)),
            pl.BlockSpec((_D, _E), lambda i, e: (0, 0)),
            pl.BlockSpec((1, _D, _F), lambda i, e: (e, 0, 0)),
            pl.BlockSpec((1, 1, _F), lambda i, e: (e, 0, 0)),
            pl.BlockSpec((1, _F, _D), lambda i, e: (e, 0, 0)),
            pl.BlockSpec((1, 1, _D), lambda i, e: (e, 0, 0)),
        ],
        out_specs=pl.BlockSpec((_BT, _D), lambda i, e: (i, 0)),
        out_shape=jax.ShapeDtypeStruct((_T, _D), jnp.float32),
        scratch_shapes=[pltpu.VMEM((_BT, _E), jnp.float32)],
        compiler_params=pltpu.CompilerParams(
            dimension_semantics=("arbitrary", "arbitrary")),
    )(x, wg.astype(jnp.bfloat16), w1, b1.reshape(_E, 1, _F), w2,
      b2.reshape(_E, 1, _D))


@jax.jit
def kernel(x, Wg0, W1_0, b1_0, W2_0, b2_0, Wg1, W1_1, b1_1, W2_1, b2_1):
    h = _moe_block(x, Wg0, W1_0, b1_0, W2_0, b2_0)
    return _moe_block(h, Wg1, W1_1, b1_1, W2_1, b2_1)


# speedup vs baseline: 1.0780x; 1.0118x over previous
import jax
import jax.numpy as jnp
from jax.experimental import pallas as pl
from jax.experimental.pallas import tpu as pltpu

_T, _D, _E, _F, _K = 2048, 1024, 8, 1024, 4
_BT = 1024
_EP = 2


def _moe_body(x_ref, wg_ref, w1_ref, b1_ref, w2_ref, b2_ref, out_ref, g_ref):
    e = pl.program_id(1)
    xb = x_ref[...].astype(jnp.bfloat16)

    @pl.when(e == 0)
    def _():
        logits = jax.lax.dot_general(
            xb, wg_ref[...], (((1,), (0,)), ((), ())),
            preferred_element_type=jnp.float32)
        lt = jnp.transpose(logits)
        row = jax.lax.broadcasted_iota(jnp.int32, (_E, _BT), 0)
        cnt = jnp.zeros((_E, _BT), jnp.float32)
        for e2 in range(_E):
            l2 = lt[e2:e2 + 1, :]
            beats = (l2 > lt) | ((l2 == lt) & (e2 < row))
            cnt += beats.astype(jnp.float32)
        sel = cnt < float(_K)
        m = jnp.max(lt, axis=0, keepdims=True)
        z = jnp.where(sel, jnp.exp(lt - m), 0.0)
        gt = z / jnp.sum(z, axis=0, keepdims=True)
        g_ref[...] = jnp.transpose(gt)

    col = jax.lax.broadcasted_iota(jnp.int32, (_BT, _E), 1)
    contrib = None
    for j in range(_EP):
        h = jnp.dot(xb, w1_ref[j].astype(jnp.bfloat16),
                    preferred_element_type=jnp.float32)
        h = jnp.maximum(h + b1_ref[j], 0.0)
        o = jnp.dot(h.astype(jnp.bfloat16), w2_ref[j].astype(jnp.bfloat16),
                    preferred_element_type=jnp.float32)
        o = o + b2_ref[j]
        ge = jnp.sum(jnp.where(col == e * _EP + j, g_ref[...], 0.0),
                     axis=1, keepdims=True)
        contrib = ge * o if contrib is None else contrib + ge * o

    @pl.when(e == 0)
    def _():
        out_ref[...] = contrib

    @pl.when(e != 0)
    def _():
        out_ref[...] += contrib


def _moe_block(x, wg, w1, b1, w2, b2):
    return pl.pallas_call(
        _moe_body,
        grid=(_T // _BT, _E // _EP),
        in_specs=[
            pl.BlockSpec((_BT, _D), lambda i, e: (i, 0)),
            pl.BlockSpec((_D, _E), lambda i, e: (0, 0)),
            pl.BlockSpec((_EP, _D, _F), lambda i, e: (e, 0, 0)),
            pl.BlockSpec((_EP, 1, _F), lambda i, e: (e, 0, 0)),
            pl.BlockSpec((_EP, _F, _D), lambda i, e: (e, 0, 0)),
            pl.BlockSpec((_EP, 1, _D), lambda i, e: (e, 0, 0)),
        ],
        out_specs=pl.BlockSpec((_BT, _D), lambda i, e: (i, 0)),
        out_shape=jax.ShapeDtypeStruct((_T, _D), jnp.float32),
        scratch_shapes=[pltpu.VMEM((_BT, _E), jnp.float32)],
        compiler_params=pltpu.CompilerParams(
            dimension_semantics=("arbitrary", "arbitrary")),
    )(x, wg.astype(jnp.bfloat16), w1, b1.reshape(_E, 1, _F), w2,
      b2.reshape(_E, 1, _D))


@jax.jit
def kernel(x, Wg0, W1_0, b1_0, W2_0, b2_0, Wg1, W1_1, b1_1, W2_1, b2_1):
    h = _moe_block(x, Wg0, W1_0, b1_0, W2_0, b2_0)
    return _moe_block(h, Wg1, W1_1, b1_1, W2_1, b2_1)


# speedup vs baseline: 1.1004x; 1.0208x over previous
import jax
import jax.numpy as jnp
from jax.experimental import pallas as pl
from jax.experimental.pallas import tpu as pltpu

_T, _D, _E, _F, _K = 2048, 1024, 8, 1024, 4
_BT = 1024
_EP = 2


def _moe_body(x_ref, wg_ref, w1_ref, b1_ref, w2_ref, b2_ref, out_ref,
              g_ref, xb_ref):
    e = pl.program_id(1)

    @pl.when(e == 0)
    def _():
        xb_ref[...] = x_ref[...].astype(jnp.bfloat16)
        logits = jax.lax.dot_general(
            xb_ref[...], wg_ref[...], (((1,), (0,)), ((), ())),
            preferred_element_type=jnp.float32)
        lt = jnp.transpose(logits)
        row = jax.lax.broadcasted_iota(jnp.int32, (_E, _BT), 0)
        cnt = jnp.zeros((_E, _BT), jnp.float32)
        for e2 in range(_E):
            l2 = lt[e2:e2 + 1, :]
            beats = (l2 > lt) | ((l2 == lt) & (e2 < row))
            cnt += beats.astype(jnp.float32)
        sel = cnt < float(_K)
        m = jnp.max(lt, axis=0, keepdims=True)
        z = jnp.where(sel, jnp.exp(lt - m), 0.0)
        gt = z / jnp.sum(z, axis=0, keepdims=True)
        g_ref[...] = jnp.transpose(gt)

    col = jax.lax.broadcasted_iota(jnp.int32, (_BT, _E), 1)
    contrib = None
    for j in range(_EP):
        h = jnp.dot(xb_ref[...], w1_ref[j].astype(jnp.bfloat16),
                    preferred_element_type=jnp.float32)
        h = jnp.maximum(h + b1_ref[j], 0.0)
        o = jnp.dot(h.astype(jnp.bfloat16), w2_ref[j].astype(jnp.bfloat16),
                    preferred_element_type=jnp.float32)
        o = o + b2_ref[j]
        ge = jnp.sum(jnp.where(col == e * _EP + j, g_ref[...], 0.0),
                     axis=1, keepdims=True)
        contrib = ge * o if contrib is None else contrib + ge * o

    @pl.when(e == 0)
    def _():
        out_ref[...] = contrib

    @pl.when(e != 0)
    def _():
        out_ref[...] += contrib


def _moe_block(x, wg, w1, b1, w2, b2):
    return pl.pallas_call(
        _moe_body,
        grid=(_T // _BT, _E // _EP),
        in_specs=[
            pl.BlockSpec((_BT, _D), lambda i, e: (i, 0)),
            pl.BlockSpec((_D, _E), lambda i, e: (0, 0)),
            pl.BlockSpec((_EP, _D, _F), lambda i, e: (e, 0, 0)),
            pl.BlockSpec((_EP, 1, _F), lambda i, e: (e, 0, 0)),
            pl.BlockSpec((_EP, _F, _D), lambda i, e: (e, 0, 0)),
            pl.BlockSpec((_EP, 1, _D), lambda i, e: (e, 0, 0)),
        ],
        out_specs=pl.BlockSpec((_BT, _D), lambda i, e: (i, 0)),
        out_shape=jax.ShapeDtypeStruct((_T, _D), jnp.float32),
        scratch_shapes=[pltpu.VMEM((_BT, _E), jnp.float32),
                        pltpu.VMEM((_BT, _D), jnp.bfloat16)],
        compiler_params=pltpu.CompilerParams(
            dimension_semantics=("arbitrary", "arbitrary")),
    )(x, wg.astype(jnp.bfloat16), w1, b1.reshape(_E, 1, _F), w2,
      b2.reshape(_E, 1, _D))


@jax.jit
def kernel(x, Wg0, W1_0, b1_0, W2_0, b2_0, Wg1, W1_1, b1_1, W2_1, b2_1):
    h = _moe_block(x, Wg0, W1_0, b1_0, W2_0, b2_0)
    return _moe_block(h, Wg1, W1_1, b1_1, W2_1, b2_1)
